# baseline (speedup 1.0000x reference)
import functools
import math

import jax
import jax.numpy as jnp
from jax import lax
from jax.experimental import pallas as pl
from jax.experimental.pallas import tpu as pltpu
from jax.experimental.pallas import tpu_sc as plsc

HS = 768
GH = 256
NH = 8
MAXN = 512
DH = GH // NH
KK = MAXN // 2
NB = 2
NROWS = NB * MAXN

SC_NC = 2
SC_NS = 16
SC_L = 16
SC_NW = SC_NC * SC_NS
ROWS_PER_W = NROWS // SC_NW


def _attn_body(hs_ref, wpt_ref, bp_ref, wqt_ref, bq_ref, wkt_ref, bk_ref,
               gamma_ref, beta_ref, nf_ref, avg_ref, thr_ref, need_ref):
    hs = hs_ref[...]
    node = jnp.dot(hs, wpt_ref[...],
                   preferred_element_type=jnp.float32) + bp_ref[...]
    mu = jnp.mean(node, axis=-1, keepdims=True)
    var = jnp.mean((node - mu) ** 2, axis=-1, keepdims=True)
    nf_ref[...] = ((node - mu) / jnp.sqrt(var + 1e-5)) * gamma_ref[...] \
        + beta_ref[...]
    q = jnp.dot(node, wqt_ref[...],
                preferred_element_type=jnp.float32) + bq_ref[...]
    k = jnp.dot(node, wkt_ref[...],
                preferred_element_type=jnp.float32) + bk_ref[...]
    scale = 1.0 / math.sqrt(DH)
    acc = jnp.zeros((MAXN, MAXN), jnp.float32)
    for h in range(NH):
        qh = q[:, h * DH:(h + 1) * DH]
        kh = k[:, h * DH:(h + 1) * DH]
        s = lax.dot_general(qh, kh, (((1,), (1,)), ((), ())),
                            preferred_element_type=jnp.float32) * scale
        m = jnp.max(s, axis=-1, keepdims=True)
        e = jnp.exp(s - m)
        acc = acc + e / jnp.sum(e, axis=-1, keepdims=True)
    avg = acc * (1.0 / NH)
    avg_ref[...] = avg
    bits = lax.bitcast_convert_type(avg, jnp.int32)

    def bisect(i, t):
        cand = t | jnp.left_shift(jnp.int32(1), 29 - i)
        cnt = jnp.sum((bits >= cand).astype(jnp.int32), axis=-1,
                      keepdims=True)
        return jnp.where(cnt >= KK, cand, t)

    t = lax.fori_loop(0, 30, bisect, jnp.zeros((MAXN, 1), jnp.int32))
    thr = lax.bitcast_convert_type(t, jnp.float32)
    ngt = jnp.sum((avg > thr).astype(jnp.int32), axis=-1, keepdims=True)
    thr_ref[...] = thr
    need_ref[...] = KK - ngt


def _attn_call(hs, wpt, bp, wqt, bq, wkt, bk, gamma, beta):
    return pl.pallas_call(
        _attn_body,
        grid=(NB,),
        in_specs=[
            pl.BlockSpec((MAXN, HS), lambda b: (b, 0)),
            pl.BlockSpec((HS, GH), lambda b: (0, 0)),
            pl.BlockSpec((1, GH), lambda b: (0, 0)),
            pl.BlockSpec((GH, GH), lambda b: (0, 0)),
            pl.BlockSpec((1, GH), lambda b: (0, 0)),
            pl.BlockSpec((GH, GH), lambda b: (0, 0)),
            pl.BlockSpec((1, GH), lambda b: (0, 0)),
            pl.BlockSpec((1, GH), lambda b: (0, 0)),
            pl.BlockSpec((1, GH), lambda b: (0, 0)),
        ],
        out_specs=[
            pl.BlockSpec((MAXN, GH), lambda b: (b, 0)),
            pl.BlockSpec((MAXN, MAXN), lambda b: (b, 0)),
            pl.BlockSpec((MAXN, 1), lambda b: (b, 0)),
            pl.BlockSpec((MAXN, 1), lambda b: (b, 0)),
        ],
        out_shape=[
            jax.ShapeDtypeStruct((NROWS, GH), jnp.float32),
            jax.ShapeDtypeStruct((NROWS, MAXN), jnp.float32),
            jax.ShapeDtypeStruct((NROWS, 1), jnp.float32),
            jax.ShapeDtypeStruct((NROWS, 1), jnp.int32),
        ],
    )(hs, wpt, bp, wqt, bq, wkt, bk, gamma, beta)


def _topk_sc_body(avg_hbm, thr_hbm, need_hbm, dst_hbm, att_hbm,
                  thr_v, need_v, rows_v, dst_all, att_all):
    wid = lax.axis_index("s") * SC_NC + lax.axis_index("c")
    base = wid * ROWS_PER_W
    pltpu.sync_copy(thr_hbm.at[pl.ds(base, ROWS_PER_W)], thr_v)
    pltpu.sync_copy(need_hbm.at[pl.ds(base, ROWS_PER_W)], need_v)
    pltpu.sync_copy(avg_hbm.at[pl.ds(base * MAXN, ROWS_PER_W * MAXN)], rows_v)
    lane = lax.iota(jnp.int32, SC_L)

    def row_body(r, carry):
        rix = jnp.full((SC_L,), r, jnp.int32)
        thr_b = plsc.load_gather(thr_v, [rix])
        need_b = plsc.load_gather(need_v, [rix])
        row = base + r
        off_row = jnp.where(row >= MAXN, MAXN, 0)
        rbase = r * MAXN
        obase = r * KK
        eq_cnt = jnp.int32(0)
        out_off = jnp.int32(0)
        for j in range(MAXN // SC_L):
            v = rows_v[pl.ds(rbase + j * SC_L, SC_L)]
            gt = v > thr_b
            eq = v == thr_b
            eq_i = eq.astype(jnp.int32)
            eq_cum = plsc.cumsum(eq_i)
            sel = jnp.logical_or(
                gt, jnp.logical_and(eq, (eq_cum - eq_i + eq_cnt) < need_b))
            sel_i = sel.astype(jnp.int32)
            sel_cum = plsc.cumsum(sel_i)
            pos = jnp.minimum(out_off + sel_cum - sel_i, KK - 1) + obase
            idxv = lane + (j * SC_L + off_row)
            plsc.store_scatter(dst_all, [pos], idxv, mask=sel)
            plsc.store_scatter(att_all, [pos], v, mask=sel)
            eq_cnt = eq_cnt + eq_cum[SC_L - 1]
            out_off = out_off + sel_cum[SC_L - 1]
        return carry

    lax.fori_loop(0, ROWS_PER_W, row_body, jnp.int32(0))
    pltpu.sync_copy(dst_all, dst_hbm.at[pl.ds(base * KK, ROWS_PER_W * KK)])
    pltpu.sync_copy(att_all, att_hbm.at[pl.ds(base * KK, ROWS_PER_W * KK)])


def _topk_sc(avg, thr, need):
    mesh = plsc.VectorSubcoreMesh(core_axis_name="c", subcore_axis_name="s")
    return pl.kernel(
        _topk_sc_body,
        out_type=(
            jax.ShapeDtypeStruct((NROWS * KK,), jnp.int32),
            jax.ShapeDtypeStruct((NROWS * KK,), jnp.float32),
        ),
        mesh=mesh,
        compiler_params=pltpu.CompilerParams(needs_layout_passes=False),
        scratch_types=[
            pltpu.VMEM((ROWS_PER_W,), jnp.float32),
            pltpu.VMEM((ROWS_PER_W,), jnp.int32),
            pltpu.VMEM((ROWS_PER_W * MAXN,), jnp.float32),
            pltpu.VMEM((ROWS_PER_W * KK,), jnp.int32),
            pltpu.VMEM((ROWS_PER_W * KK,), jnp.float32),
        ],
    )(avg.reshape(-1), thr, need)


EDGE_R = 32


def _edge_body(att_ref, we0_ref, be_ref, src_ref, ea_ref):
    r0 = pl.program_id(0) * EDGE_R
    src_ref[...] = r0 + lax.broadcasted_iota(jnp.int32, (EDGE_R, KK), 0)
    a = att_ref[...][:, :, None]
    ea_ref[...] = a * we0_ref[...][None, :, :] + be_ref[...][None, :, :]


def _edge_call(att, we0, be):
    return pl.pallas_call(
        _edge_body,
        grid=(NROWS // EDGE_R,),
        in_specs=[
            pl.BlockSpec((EDGE_R, KK), lambda i: (i, 0)),
            pl.BlockSpec((1, GH), lambda i: (0, 0)),
            pl.BlockSpec((1, GH), lambda i: (0, 0)),
        ],
        out_specs=[
            pl.BlockSpec((EDGE_R, KK), lambda i: (i, 0)),
            pl.BlockSpec((EDGE_R, KK, GH), lambda i: (i, 0, 0)),
        ],
        out_shape=[
            jax.ShapeDtypeStruct((NROWS, KK), jnp.int32),
            jax.ShapeDtypeStruct((NROWS, KK, GH), jnp.float32),
        ],
    )(att, we0, be)


def kernel(hidden_states, attention_mask, Wp, bp, W_in, b_in, W_out, b_out,
           We, be, gamma, beta):
    B = hidden_states.shape[0]
    hs = hidden_states[:, :MAXN, :].reshape(B * MAXN, HS)
    nf, avg, thr, need = _attn_call(
        hs, Wp.T, bp.reshape(1, GH),
        W_in[:GH].T, b_in[:GH].reshape(1, GH),
        W_in[GH:2 * GH].T, b_in[GH:2 * GH].reshape(1, GH),
        gamma.reshape(1, GH), beta.reshape(1, GH))
    dst, att = _topk_sc(avg, thr.reshape(-1), need.reshape(-1))
    src, ea = _edge_call(att.reshape(NROWS, KK), We[:, 0].reshape(1, GH),
                         be.reshape(1, GH))
    edge_index = jnp.stack([src.reshape(-1), dst])
    return (nf.reshape(B, MAXN, GH), edge_index,
            ea.reshape(NROWS * KK, GH))


# speedup vs baseline: 1.0387x; 1.0387x over previous
import functools
import math

import jax
import jax.numpy as jnp
from jax import lax
from jax.experimental import pallas as pl
from jax.experimental.pallas import tpu as pltpu
from jax.experimental.pallas import tpu_sc as plsc

HS = 768
GH = 256
NH = 8
MAXN = 512
DH = GH // NH
KK = MAXN // 2
NB = 2
NROWS = NB * MAXN

SC_NC = 2
SC_NS = 16
SC_L = 16
SC_NW = SC_NC * SC_NS
ROWS_PER_W = NROWS // SC_NW


def _attn_body(hs_ref, wpt_ref, bp_ref, wqt_ref, bq_ref, wkt_ref, bk_ref,
               gamma_ref, beta_ref, nf_ref, avg_ref, thr_ref, need_ref):
    hs = hs_ref[...]
    node = jnp.dot(hs, wpt_ref[...],
                   preferred_element_type=jnp.float32) + bp_ref[...]
    mu = jnp.mean(node, axis=-1, keepdims=True)
    var = jnp.mean((node - mu) ** 2, axis=-1, keepdims=True)
    nf_ref[...] = ((node - mu) / jnp.sqrt(var + 1e-5)) * gamma_ref[...] \
        + beta_ref[...]
    q = jnp.dot(node, wqt_ref[...],
                preferred_element_type=jnp.float32) + bq_ref[...]
    k = jnp.dot(node, wkt_ref[...],
                preferred_element_type=jnp.float32) + bk_ref[...]
    scale = 1.0 / math.sqrt(DH)
    acc = jnp.zeros((MAXN, MAXN), jnp.float32)
    for h in range(NH):
        qh = q[:, h * DH:(h + 1) * DH]
        kh = k[:, h * DH:(h + 1) * DH]
        s = lax.dot_general(qh, kh, (((1,), (1,)), ((), ())),
                            preferred_element_type=jnp.float32) * scale
        m = jnp.max(s, axis=-1, keepdims=True)
        e = jnp.exp(s - m)
        acc = acc + e / jnp.sum(e, axis=-1, keepdims=True)
    avg = acc * (1.0 / NH)
    avg_ref[...] = avg
    bits = lax.bitcast_convert_type(avg, jnp.int32)

    def bisect(i, t):
        cand = t | jnp.left_shift(jnp.int32(1), 29 - i)
        cnt = jnp.sum((bits >= cand).astype(jnp.int32), axis=-1,
                      keepdims=True)
        return jnp.where(cnt >= KK, cand, t)

    t = lax.fori_loop(0, 30, bisect, jnp.zeros((MAXN, 1), jnp.int32))
    thr = lax.bitcast_convert_type(t, jnp.float32)
    ngt = jnp.sum((avg > thr).astype(jnp.int32), axis=-1, keepdims=True)
    thr_ref[...] = thr
    need_ref[...] = KK - ngt


def _attn_call(hs, wpt, bp, wqt, bq, wkt, bk, gamma, beta):
    return pl.pallas_call(
        _attn_body,
        grid=(NB,),
        in_specs=[
            pl.BlockSpec((MAXN, HS), lambda b: (b, 0)),
            pl.BlockSpec((HS, GH), lambda b: (0, 0)),
            pl.BlockSpec((1, GH), lambda b: (0, 0)),
            pl.BlockSpec((GH, GH), lambda b: (0, 0)),
            pl.BlockSpec((1, GH), lambda b: (0, 0)),
            pl.BlockSpec((GH, GH), lambda b: (0, 0)),
            pl.BlockSpec((1, GH), lambda b: (0, 0)),
            pl.BlockSpec((1, GH), lambda b: (0, 0)),
            pl.BlockSpec((1, GH), lambda b: (0, 0)),
        ],
        out_specs=[
            pl.BlockSpec((MAXN, GH), lambda b: (b, 0)),
            pl.BlockSpec((MAXN, MAXN), lambda b: (b, 0)),
            pl.BlockSpec((MAXN, 1), lambda b: (b, 0)),
            pl.BlockSpec((MAXN, 1), lambda b: (b, 0)),
        ],
        out_shape=[
            jax.ShapeDtypeStruct((NROWS, GH), jnp.float32),
            jax.ShapeDtypeStruct((NROWS, MAXN), jnp.float32),
            jax.ShapeDtypeStruct((NROWS, 1), jnp.float32),
            jax.ShapeDtypeStruct((NROWS, 1), jnp.int32),
        ],
    )(hs, wpt, bp, wqt, bq, wkt, bk, gamma, beta)


def _topk_sc_body(avg_hbm, thr_hbm, need_hbm, dst_hbm, att_hbm,
                  thr_v, need_v, rows_v, dst_all, att_all):
    wid = lax.axis_index("s") * SC_NC + lax.axis_index("c")
    base = wid * ROWS_PER_W
    pltpu.sync_copy(thr_hbm.at[pl.ds(base, ROWS_PER_W)], thr_v)
    pltpu.sync_copy(need_hbm.at[pl.ds(base, ROWS_PER_W)], need_v)
    pltpu.sync_copy(avg_hbm.at[pl.ds(base * MAXN, ROWS_PER_W * MAXN)], rows_v)
    lane = lax.iota(jnp.int32, SC_L)

    UNROLL = 8
    for g in range(ROWS_PER_W // SC_L):
        lrow = g * SC_L + lane
        thr_b = thr_v[pl.ds(g * SC_L, SC_L)]
        need_b = need_v[pl.ds(g * SC_L, SC_L)]
        off_row = jnp.where((base + lrow) >= MAXN, MAXN, 0)
        gbase = lrow * MAXN
        obase = lrow * KK
        zero = jnp.zeros((SC_L,), jnp.int32)

        def col_body(i, carry):
            eq_cnt, out_off = carry
            for u in range(UNROLL):
                c = i * UNROLL + u
                v = plsc.load_gather(rows_v, [gbase + c])
                gt = v > thr_b
                eq = v == thr_b
                sel = jnp.logical_or(
                    gt, jnp.logical_and(eq, eq_cnt < need_b))
                pos = obase + jnp.minimum(out_off, KK - 1)
                plsc.store_scatter(dst_all, [pos], off_row + c, mask=sel)
                plsc.store_scatter(att_all, [pos], v, mask=sel)
                eq_cnt = eq_cnt + eq.astype(jnp.int32)
                out_off = out_off + sel.astype(jnp.int32)
            return eq_cnt, out_off

        lax.fori_loop(0, MAXN // UNROLL, col_body, (zero, zero))
    pltpu.sync_copy(dst_all, dst_hbm.at[pl.ds(base * KK, ROWS_PER_W * KK)])
    pltpu.sync_copy(att_all, att_hbm.at[pl.ds(base * KK, ROWS_PER_W * KK)])


def _topk_sc(avg, thr, need):
    mesh = plsc.VectorSubcoreMesh(core_axis_name="c", subcore_axis_name="s")
    return pl.kernel(
        _topk_sc_body,
        out_type=(
            jax.ShapeDtypeStruct((NROWS * KK,), jnp.int32),
            jax.ShapeDtypeStruct((NROWS * KK,), jnp.float32),
        ),
        mesh=mesh,
        compiler_params=pltpu.CompilerParams(needs_layout_passes=False),
        scratch_types=[
            pltpu.VMEM((ROWS_PER_W,), jnp.float32),
            pltpu.VMEM((ROWS_PER_W,), jnp.int32),
            pltpu.VMEM((ROWS_PER_W * MAXN,), jnp.float32),
            pltpu.VMEM((ROWS_PER_W * KK,), jnp.int32),
            pltpu.VMEM((ROWS_PER_W * KK,), jnp.float32),
        ],
    )(avg.reshape(-1), thr, need)


EDGE_R = 32


def _edge_body(att_ref, we0_ref, be_ref, src_ref, ea_ref):
    r0 = pl.program_id(0) * EDGE_R
    src_ref[...] = r0 + lax.broadcasted_iota(jnp.int32, (EDGE_R, KK), 0)
    a = att_ref[...][:, :, None]
    ea_ref[...] = a * we0_ref[...][None, :, :] + be_ref[...][None, :, :]


def _edge_call(att, we0, be):
    return pl.pallas_call(
        _edge_body,
        grid=(NROWS // EDGE_R,),
        in_specs=[
            pl.BlockSpec((EDGE_R, KK), lambda i: (i, 0)),
            pl.BlockSpec((1, GH), lambda i: (0, 0)),
            pl.BlockSpec((1, GH), lambda i: (0, 0)),
        ],
        out_specs=[
            pl.BlockSpec((EDGE_R, KK), lambda i: (i, 0)),
            pl.BlockSpec((EDGE_R, KK, GH), lambda i: (i, 0, 0)),
        ],
        out_shape=[
            jax.ShapeDtypeStruct((NROWS, KK), jnp.int32),
            jax.ShapeDtypeStruct((NROWS, KK, GH), jnp.float32),
        ],
    )(att, we0, be)


def kernel(hidden_states, attention_mask, Wp, bp, W_in, b_in, W_out, b_out,
           We, be, gamma, beta):
    B = hidden_states.shape[0]
    hs = hidden_states[:, :MAXN, :].reshape(B * MAXN, HS)
    nf, avg, thr, need = _attn_call(
        hs, Wp.T, bp.reshape(1, GH),
        W_in[:GH].T, b_in[:GH].reshape(1, GH),
        W_in[GH:2 * GH].T, b_in[GH:2 * GH].reshape(1, GH),
        gamma.reshape(1, GH), beta.reshape(1, GH))
    dst, att = _topk_sc(avg, thr.reshape(-1), need.reshape(-1))
    src, ea = _edge_call(att.reshape(NROWS, KK), We[:, 0].reshape(1, GH),
                         be.reshape(1, GH))
    edge_index = jnp.stack([src.reshape(-1), dst])
    return (nf.reshape(B, MAXN, GH), edge_index,
            ea.reshape(NROWS * KK, GH))
